# baseline (speedup 1.0000x reference)
import functools

import jax
import jax.numpy as jnp
from jax import lax
from jax.experimental import pallas as pl
from jax.experimental.pallas import tpu as pltpu
from jax.experimental.pallas import tpu_sc as plsc

N_USERS = 1000000
N_ITEMS = 1000000
EMBED_DIM = 32
BATCH = 16384

NUM_CORES = 2
NUM_SUBCORES = 16
NUM_WORKERS = NUM_CORES * NUM_SUBCORES
B_PER_W = BATCH // NUM_WORKERS
CHUNK = 128
N_CHUNKS = B_PER_W // CHUNK


def _tec_kernel(uid_hbm, iid_hbm, ue_hbm, ie_hbm, ub_hbm, ib_hbm, out_hbm,
                uidx_v, iidx_v, ue_v, ie_v, ub_v, ib_v, out_v, ps_v, sem):
    wid = lax.axis_index("s") * NUM_CORES + lax.axis_index("c")
    base = wid * B_PER_W
    row0 = wid * N_CHUNKS

    pltpu.sync_copy(uid_hbm.at[pl.ds(row0, N_CHUNKS)], uidx_v)
    pltpu.sync_copy(iid_hbm.at[pl.ds(row0, N_CHUNKS)], iidx_v)

    copies = []
    for j in range(N_CHUNKS):
        copies.append(pltpu.async_copy(
            ue_hbm.at[uidx_v.at[j]], ue_v.at[pl.ds(j * CHUNK, CHUNK)], sem))
        copies.append(pltpu.async_copy(
            ie_hbm.at[iidx_v.at[j]], ie_v.at[pl.ds(j * CHUNK, CHUNK)], sem))
        copies.append(pltpu.async_copy(
            ub_hbm.at[uidx_v.at[j]], ub_v.at[pl.ds(j * CHUNK, CHUNK)], sem))
        copies.append(pltpu.async_copy(
            ib_hbm.at[iidx_v.at[j]], ib_v.at[pl.ds(j * CHUNK, CHUNK)], sem))
    for cp in copies:
        cp.wait()

    tidx = lax.iota(jnp.int32, 16) * 17

    def group_body(g, _):
        g16 = g * 16
        for r16 in range(16):
            row = g16 + r16
            a = (ue_v[row, pl.ds(0, 16)] * ie_v[row, pl.ds(0, 16)]
                 + ue_v[row, pl.ds(16, 16)] * ie_v[row, pl.ds(16, 16)])
            ps_v[pl.ds(r16 * 17, 16)] = a
        acc = ub_v[pl.ds(g16, 16)] + ib_v[pl.ds(g16, 16)]
        for j in range(16):
            acc = acc + plsc.load_gather(ps_v, [tidx + j])
        out_v[pl.ds(g16, 16)] = acc
        return 0

    lax.fori_loop(0, B_PER_W // 16, group_body, 0)

    pltpu.sync_copy(out_v, out_hbm.at[pl.ds(base, B_PER_W)])


@jax.jit
def _run(uid2, iid2, user_emb, item_emb, ub1, ib1):
    mesh = plsc.VectorSubcoreMesh(
        core_axis_name="c", subcore_axis_name="s",
        num_cores=NUM_CORES, num_subcores=NUM_SUBCORES)
    return pl.kernel(
        _tec_kernel,
        out_type=jax.ShapeDtypeStruct((BATCH,), jnp.float32),
        mesh=mesh,
        compiler_params=pltpu.CompilerParams(
            needs_layout_passes=False, use_tc_tiling_on_sc=False),
        scratch_types=[
            pltpu.VMEM((N_CHUNKS, CHUNK), jnp.int32),
            pltpu.VMEM((N_CHUNKS, CHUNK), jnp.int32),
            pltpu.VMEM((B_PER_W, EMBED_DIM), jnp.float32),
            pltpu.VMEM((B_PER_W, EMBED_DIM), jnp.float32),
            pltpu.VMEM((B_PER_W,), jnp.float32),
            pltpu.VMEM((B_PER_W,), jnp.float32),
            pltpu.VMEM((B_PER_W,), jnp.float32),
            pltpu.VMEM((16 * 17,), jnp.float32),
            pltpu.SemaphoreType.DMA,
        ],
    )(uid2, iid2, user_emb, item_emb, ub1, ib1)


def kernel(user_ids, item_ids, user_emb, item_emb, user_bias, item_bias):
    uid2 = user_ids.astype(jnp.int32).reshape(BATCH // CHUNK, CHUNK)
    iid2 = item_ids.astype(jnp.int32).reshape(BATCH // CHUNK, CHUNK)
    ub1 = user_bias.reshape(N_USERS)
    ib1 = item_bias.reshape(N_ITEMS)
    return _run(uid2, iid2, user_emb, item_emb, ub1, ib1)
